# Initial kernel scaffold; baseline (speedup 1.0000x reference)
import jax
import jax.numpy as jnp
from jax.experimental import pallas as pl


def kernel(img, params):
    raise NotImplementedError("write your pallas kernel here")


# speedup vs baseline: 1.0029x; 1.0029x over previous
import functools

import jax
import jax.numpy as jnp
from jax import lax
from jax.experimental import pallas as pl
from jax.experimental.pallas import tpu as pltpu


def _conv(x, w, b, stride, pad):
    y = lax.conv_general_dilated(x, w, (stride, stride), ((pad, pad), (pad, pad)),
                                 dimension_numbers=('NCHW', 'OIHW', 'NCHW'))
    return y + b[None, :, None, None]


def _convT(x, w, b, stride, pad):
    k = w.shape[2]
    wt = jnp.flip(w, (2, 3)).transpose(1, 0, 2, 3)
    q = k - 1 - pad
    y = lax.conv_general_dilated(x, wt, (1, 1), ((q, q), (q, q)),
                                 lhs_dilation=(stride, stride),
                                 dimension_numbers=('NCHW', 'OIHW', 'NCHW'))
    return y + b[None, :, None, None]


def _resblock(x, w1, b1, w2, b2):
    h = jax.nn.relu(x)
    h = _conv(h, w1, b1, 1, 1)
    h = jax.nn.relu(h)
    h = _conv(h, w2, b2, 1, 0)
    return x + h


def _encoder(x, p):
    x = jax.nn.relu(_conv(x, p['e_c1_w'], p['e_c1_b'], 2, 1))
    x = jax.nn.relu(_conv(x, p['e_c2_w'], p['e_c2_b'], 2, 1))
    x = _conv(x, p['e_c3_w'], p['e_c3_b'], 1, 1)
    for i in range(2):
        x = _resblock(x, p['e_rb%d_w1' % i], p['e_rb%d_b1' % i],
                      p['e_rb%d_w2' % i], p['e_rb%d_b2' % i])
    x = jax.nn.relu(x)
    x = _conv(x, p['e_out_w'], p['e_out_b'], 1, 0)
    return x


def _decoder(x, p):
    x = _conv(x, p['d_c1_w'], p['d_c1_b'], 1, 1)
    for i in range(2):
        x = _resblock(x, p['d_rb%d_w1' % i], p['d_rb%d_b1' % i],
                      p['d_rb%d_w2' % i], p['d_rb%d_b2' % i])
    x = jax.nn.relu(x)
    x = _convT(x, p['d_t1_w'], p['d_t1_b'], 2, 1)
    x = jax.nn.relu(x)
    x = _convT(x, p['d_t2_w'], p['d_t2_b'], 2, 1)
    return x


_NUM_CODES = 512


def _vq_body(x_ref, embt_ref, emb_ref, out_ref):
    x = x_ref[...]
    embt = embt_ref[...]
    m = jax.lax.dot_general(x.astype(jnp.bfloat16), embt.astype(jnp.bfloat16),
                            (((1,), (0,)), ((), ())),
                            preferred_element_type=jnp.float32)
    sx = jnp.sum(x * x, axis=1, keepdims=True)
    se = jnp.sum(embt * embt, axis=0, keepdims=True)
    d2 = (sx + se) - 2.0 * m
    dis = jnp.sqrt(jnp.maximum(d2, 0.0))
    minv = jnp.min(dis, axis=1, keepdims=True)
    iota = jax.lax.broadcasted_iota(jnp.int32, dis.shape, 1)
    idx = jnp.min(jnp.where(dis == minv, iota, _NUM_CODES), axis=1)
    onehot = (iota == idx[:, None]).astype(jnp.float32)
    q = jax.lax.dot_general(onehot, emb_ref[...], (((1,), (0,)), ((), ())),
                            precision=jax.lax.Precision.HIGHEST,
                            preferred_element_type=jnp.float32)
    out_ref[...] = x + (q - x)


@functools.partial(jax.jit, static_argnames=('tile_m',))
def _vq_pallas(xf, emb, tile_m=512):
    M, C = xf.shape
    K = emb.shape[0]
    embt = emb.T
    grid = (M // tile_m,)
    return pl.pallas_call(
        _vq_body,
        grid=grid,
        in_specs=[
            pl.BlockSpec((tile_m, C), lambda i: (i, 0)),
            pl.BlockSpec((C, K), lambda i: (0, 0)),
            pl.BlockSpec((K, C), lambda i: (0, 0)),
        ],
        out_specs=pl.BlockSpec((tile_m, C), lambda i: (i, 0)),
        out_shape=jax.ShapeDtypeStruct((M, C), jnp.float32),
    )(xf, embt, emb)


def kernel(img, params):
    x = _encoder(img, params)
    B, C, H, W = x.shape
    xf = x.transpose(0, 2, 3, 1).reshape(-1, C)
    qx = _vq_pallas(xf, params['embedding'])
    x = qx.reshape(B, H, W, C).transpose(0, 3, 1, 2)
    return _decoder(x, params)
